# baseline (device time: 42910 ns/iter reference)
import jax
import jax.numpy as jnp
from jax import lax
from jax.experimental import pallas as pl
from jax.experimental.pallas import tpu as pltpu

N_DEV = 8
MX, MY, MZ = 1, 3, 4

PARTS = (
    (0, 96, (MX, MY, MZ)),
    (96, 80, (MY, MZ, MX)),
    (176, 80, (MZ, MX, MY)),
)


def kernel(A, B):
    m, _ = A.shape
    _, n = B.shape
    chunk = m // N_DEV

    def body(a_ref, b_ref, out_ref,
             snd0, rcv0, snd1, rcv1, snd2, rcv2, b16,
             ss0, rs0, ss1, rs1, ss2, rs2):
        d = lax.axis_index("i")
        snd = (snd0, snd1, snd2)
        rcv = (rcv0, rcv1, rcv2)
        ssem = (ss0, ss1, ss2)
        rsem = (rs0, rs1, rs2)

        b16[...] = b_ref[...].astype(jnp.bfloat16)

        def dot_part(mask, row0, nrows, out_dtype=jnp.bfloat16):
            c = jnp.bitwise_xor(d, mask)
            p = jnp.dot(
                a_ref[pl.ds(c * chunk + row0, nrows), :].astype(jnp.bfloat16),
                b16[...],
                preferred_element_type=jnp.float32,
            )
            return p.astype(out_dtype) if out_dtype != jnp.float32 else p

        barrier_sem = pltpu.get_barrier_semaphore()
        for mask in (MX, MY, MZ):
            pl.semaphore_signal(
                barrier_sem, inc=1,
                device_id=(jnp.bitwise_xor(d, mask),),
                device_id_type=pl.DeviceIdType.MESH,
            )
        pl.semaphore_wait(barrier_sem, 3)

        rdmas = {}

        def exchange(p, mask, src_ref, dst_ref, sem_idx):
            rd = pltpu.make_async_remote_copy(
                src_ref=src_ref, dst_ref=dst_ref,
                send_sem=ssem[p].at[sem_idx], recv_sem=rsem[p].at[sem_idx],
                device_id=(jnp.bitwise_xor(d, mask),),
                device_id_type=pl.DeviceIdType.MESH,
            )
            rdmas[(p, sem_idx)] = rd
            rd.start()

        def gs(masks):
            _, M2, M3 = masks
            return (0, M2, M3, M2 ^ M3)

        for j in (1, 3, 2, 0):
            for p, (row0, nrows, masks) in enumerate(PARTS):
                M1 = masks[0]
                g = gs(masks)[j]
                snd[p][j] = dot_part(M1 ^ g, row0, nrows)
                exchange(p, M1, snd[p].at[j], rcv[p].at[j], j)

        for j in (1, 3):
            for p, (row0, nrows, masks) in enumerate(PARTS):
                v = dot_part(gs(masks)[j], row0, nrows)
                rdmas[(p, j)].wait_recv()
                rcv[p][j] = rcv[p][j] + v
        for p, (_, _, masks) in enumerate(PARTS):
            M2 = masks[1]
            exchange(p, M2, rcv[p].at[3], rcv[p].at[5], 5)
            exchange(p, M2, rcv[p].at[1], rcv[p].at[4], 4)
        for p, (row0, nrows, masks) in enumerate(PARTS):
            v = dot_part(gs(masks)[2], row0, nrows)
            rdmas[(p, 2)].wait_recv()
            rdmas[(p, 5)].wait_recv()
            rcv[p][2] = rcv[p][2] + v + rcv[p][5]
            exchange(p, masks[2], rcv[p].at[2], rcv[p].at[6], 6)
        for p, (row0, nrows, masks) in enumerate(PARTS):
            v = dot_part(gs(masks)[0], row0, nrows)
            rdmas[(p, 0)].wait_recv()
            rdmas[(p, 4)].wait_recv()
            rcv[p][0] = rcv[p][0] + v + rcv[p][4]

        for p, (row0, nrows, _) in enumerate(PARTS):
            rdmas[(p, 6)].wait_recv()
            out_ref[pl.ds(row0, nrows), :] = (
                rcv[p][0].astype(jnp.float32) + rcv[p][6].astype(jnp.float32)
            )

        for p in range(3):
            for i in range(7):
                rdmas[(p, i)].wait_send()

    scratch = []
    for row0, nrows, masks in PARTS:
        scratch.append(pltpu.VMEM((4, nrows, n), jnp.bfloat16))
        scratch.append(pltpu.VMEM((7, nrows, n), jnp.bfloat16))
    scratch.append(pltpu.VMEM((B.shape[0], n), jnp.bfloat16))
    for _ in range(3):
        scratch.append(pltpu.SemaphoreType.DMA((7,)))
        scratch.append(pltpu.SemaphoreType.DMA((7,)))

    return pl.pallas_call(
        body,
        out_shape=jax.ShapeDtypeStruct((chunk, n), jnp.float32),
        in_specs=[
            pl.BlockSpec(memory_space=pltpu.VMEM),
            pl.BlockSpec(memory_space=pltpu.VMEM),
        ],
        out_specs=pl.BlockSpec(memory_space=pltpu.VMEM),
        scratch_shapes=scratch,
        compiler_params=pltpu.CompilerParams(collective_id=0),
    )(A, B)
